# Initial kernel scaffold; baseline (speedup 1.0000x reference)
import jax
import jax.numpy as jnp
from jax.experimental import pallas as pl


def kernel(features, edge_index, eps0, W1_0, b1_0, W2_0, b2_0, eps1, W1_1, b1_1, W2_1, b2_1, eps2, W1_2, b1_2, W2_2, b2_2, Wc1, bc1, Wc2, bc2):
    raise NotImplementedError("write your pallas kernel here")


# speedup vs baseline: 5.7806x; 5.7806x over previous
import functools

import jax
import jax.numpy as jnp
from jax import lax
from jax.experimental import pallas as pl
from jax.experimental.pallas import tpu as pltpu
from jax.experimental.pallas import tpu_sc as plsc

N = 10000
E = 320000
D = 128
C = 10

NC = 2
NS = 16
CH = 80
NCHUNK = E // (NC * NS * CH)
NPAD = 10240
NPT = NPAD // NS

_mesh = plsc.VectorSubcoreMesh(core_axis_name="c", subcore_axis_name="s")


@functools.partial(
    pl.kernel,
    out_type=jax.ShapeDtypeStruct((NC, NPAD, D), jnp.float32),
    mesh=_mesh,
    scratch_types=[
        pltpu.VMEM((NCHUNK, CH), jnp.int32),
        pltpu.VMEM((NCHUNK, CH), jnp.int32),
        pltpu.VMEM((CH, D), jnp.float32),
        pltpu.VMEM_SHARED((NPAD, D), jnp.float32),
        pltpu.SemaphoreType.DMA,
    ],
)
def _agg(h_hbm, src_hbm, dst_hbm, zeros_hbm, out_hbm,
         src_v, dst_v, buf, acc, sem):
    c = lax.axis_index("c")
    s = lax.axis_index("s")
    pltpu.sync_copy(src_hbm.at[c, s], src_v)
    pltpu.sync_copy(dst_hbm.at[c, s], dst_v)
    pltpu.sync_copy(zeros_hbm, acc.at[pl.ds(s * NPT, NPT)])
    plsc.subcore_barrier()

    @pl.loop(0, NCHUNK)
    def _(i):
        pltpu.async_copy(h_hbm.at[src_v.at[i]], buf, sem).wait()
        pltpu.sync_copy(buf, acc.at[dst_v.at[i]], add=True)

    plsc.subcore_barrier()
    pltpu.sync_copy(acc.at[pl.ds(s * NPT, NPT)],
                    out_hbm.at[c].at[pl.ds(s * NPT, NPT)])


BR = 1000
_GRID = N // BR


def _dot_t(x, w):
    return lax.dot_general(x, w, (((1,), (1,)), ((), ())),
                           preferred_element_type=jnp.float32,
                           precision=lax.Precision.HIGHEST)


def _mlp_body(eps_ref, h_ref, a0_ref, a1_ref, w1_ref, b1_ref, w2_ref, b2_ref,
              o_ref):
    z = (1.0 + eps_ref[0]) * h_ref[...] + a0_ref[...] + a1_ref[...]
    z = jnp.maximum(_dot_t(z, w1_ref[...]) + b1_ref[...], 0.0)
    z = _dot_t(z, w2_ref[...]) + b2_ref[...]
    o_ref[...] = jnp.maximum(z, 0.0)


def _mlp(h, a0, a1, eps, W1, b1, W2, b2):
    full = lambda shape: pl.BlockSpec(shape, lambda i: (0,) * len(shape))
    row = pl.BlockSpec((BR, D), lambda i: (i, 0))
    return pl.pallas_call(
        _mlp_body,
        grid=(_GRID,),
        in_specs=[
            pl.BlockSpec(memory_space=pltpu.SMEM),
            row, row, row,
            full((D, D)), full((1, D)), full((D, D)), full((1, D)),
        ],
        out_specs=row,
        out_shape=jax.ShapeDtypeStruct((N, D), jnp.float32),
    )(eps.reshape(1), h, a0, a1, W1, b1.reshape(1, D), W2, b2.reshape(1, D))


def _final_body(eps_ref, h_ref, a0_ref, a1_ref, w1_ref, b1_ref, w2_ref,
                b2_ref, wc1_ref, bc1_ref, wc2_ref, bc2_ref, o_ref, acc_ref):
    i = pl.program_id(0)
    z = (1.0 + eps_ref[0]) * h_ref[...] + a0_ref[...] + a1_ref[...]
    z = jnp.maximum(_dot_t(z, w1_ref[...]) + b1_ref[...], 0.0)
    z = _dot_t(z, w2_ref[...]) + b2_ref[...]
    h3 = jnp.maximum(z, 0.0)
    part = jnp.sum(h3, axis=0, keepdims=True)

    @pl.when(i == 0)
    def _():
        acc_ref[...] = jnp.zeros_like(acc_ref)

    acc_ref[...] += part

    @pl.when(i == pl.num_programs(0) - 1)
    def _():
        hg = acc_ref[...]
        t = jnp.maximum(_dot_t(hg, wc1_ref[...]) + bc1_ref[...], 0.0)
        o_ref[...] = _dot_t(t, wc2_ref[...]) + bc2_ref[...]


def _final(h, a0, a1, eps, W1, b1, W2, b2, Wc1, bc1, Wc2, bc2):
    full = lambda shape: pl.BlockSpec(shape, lambda i: (0,) * len(shape))
    row = pl.BlockSpec((BR, D), lambda i: (i, 0))
    return pl.pallas_call(
        _final_body,
        grid=(_GRID,),
        in_specs=[
            pl.BlockSpec(memory_space=pltpu.SMEM),
            row, row, row,
            full((D, D)), full((1, D)), full((D, D)), full((1, D)),
            full((D, D)), full((1, D)), full((C, D)), full((1, C)),
        ],
        out_specs=full((1, C)),
        out_shape=jax.ShapeDtypeStruct((1, C), jnp.float32),
        scratch_shapes=[pltpu.VMEM((1, D), jnp.float32)],
    )(eps.reshape(1), h, a0, a1, W1, b1.reshape(1, D), W2, b2.reshape(1, D),
      Wc1, bc1.reshape(1, D), Wc2, bc2.reshape(1, C))


def kernel(features, edge_index,
           eps0, W1_0, b1_0, W2_0, b2_0,
           eps1, W1_1, b1_1, W2_1, b2_1,
           eps2, W1_2, b1_2, W2_2, b2_2,
           Wc1, bc1, Wc2, bc2):
    src = edge_index[0].reshape(NC, NS, NCHUNK, CH)
    dst = edge_index[1].reshape(NC, NS, NCHUNK, CH)
    zeros = jnp.zeros((NPT, D), jnp.float32)
    layers = [
        (eps0, W1_0, b1_0, W2_0, b2_0),
        (eps1, W1_1, b1_1, W2_1, b2_1),
        (eps2, W1_2, b1_2, W2_2, b2_2),
    ]
    h = features
    for li, (eps, W1, b1, W2, b2) in enumerate(layers):
        agg = _agg(h, src, dst, zeros)
        if li < 2:
            h = _mlp(h, agg[0], agg[1], eps, W1, b1, W2, b2)
        else:
            out = _final(h, agg[0], agg[1], eps, W1, b1, W2, b2,
                         Wc1, bc1, Wc2, bc2)
    return out
